# baseline (speedup 1.0000x reference)
import functools

import jax
import jax.numpy as jnp
from jax import lax
from jax.experimental import pallas as pl
from jax.experimental.pallas import tpu as pltpu
from jax.experimental.pallas import tpu_sc as plsc

_B = 4096
_D = 128
_BLK = 2048
_LOG2E = 1.4426950408889634


def _sc_gather(user_id, product_id, user_table, product_table):
    info = plsc.get_sparse_core_info()
    nw = info.num_cores * info.num_subcores
    bpw = _B // nw
    mesh = plsc.VectorSubcoreMesh(core_axis_name="c", subcore_axis_name="s")
    half = bpw // 2

    @functools.partial(
        pl.kernel,
        out_type=(
            jax.ShapeDtypeStruct((_B, _D), jnp.float32),
            jax.ShapeDtypeStruct((_B, _D), jnp.float32),
        ),
        mesh=mesh,
        scratch_types=(
            pltpu.VMEM((bpw,), jnp.int32),
            pltpu.VMEM((bpw, _D), jnp.float32),
            pltpu.VMEM((bpw,), jnp.int32),
            pltpu.VMEM((bpw, _D), jnp.float32),
            pltpu.SemaphoreType.DMA,
            pltpu.SemaphoreType.DMA,
            pltpu.SemaphoreType.DMA,
            pltpu.SemaphoreType.DMA,
        ),
    )
    def gather(uid_hbm, pid_hbm, utab_hbm, ptab_hbm, uout_hbm, pout_hbm,
               uidx, urows, pidx, prows, usem, psem, s1, s2):
        wid = lax.axis_index("s") * info.num_cores + lax.axis_index("c")
        base = wid * bpw
        ci = pltpu.async_copy(pid_hbm.at[pl.ds(base, bpw)], pidx, s1)
        cj = pltpu.async_copy(uid_hbm.at[pl.ds(base, bpw)], uidx, s2)
        ci.wait()
        cp0 = pltpu.async_copy(ptab_hbm.at[pidx.at[pl.ds(0, half)]],
                               prows.at[pl.ds(0, half)], psem)
        cp1 = pltpu.async_copy(ptab_hbm.at[pidx.at[pl.ds(half, half)]],
                               prows.at[pl.ds(half, half)], psem)
        cj.wait()
        cu0 = pltpu.async_copy(utab_hbm.at[uidx.at[pl.ds(0, half)]],
                               urows.at[pl.ds(0, half)], usem)
        cu1 = pltpu.async_copy(utab_hbm.at[uidx.at[pl.ds(half, half)]],
                               urows.at[pl.ds(half, half)], usem)
        cp0.wait()
        sp0 = pltpu.async_copy(prows.at[pl.ds(0, half)],
                               pout_hbm.at[pl.ds(base, half)], s1)
        cp1.wait()
        sp1 = pltpu.async_copy(prows.at[pl.ds(half, half)],
                               pout_hbm.at[pl.ds(base + half, half)], s2)
        cu0.wait()
        su0 = pltpu.async_copy(urows.at[pl.ds(0, half)],
                               uout_hbm.at[pl.ds(base, half)], s1)
        cu1.wait()
        su1 = pltpu.async_copy(urows.at[pl.ds(half, half)],
                               uout_hbm.at[pl.ds(base + half, half)], s2)
        sp0.wait()
        sp1.wait()
        su0.wait()
        su1.wait()

    return gather(user_id, product_id, user_table, product_table)


def _loss_body(u_ref, pblk_ref, acc_ref, s_ref):
    i = pl.program_id(0)
    j = pl.program_id(1)
    l2t = lax.dot_general(
        pblk_ref[...].astype(jnp.bfloat16),
        (u_ref[...] * _LOG2E).astype(jnp.bfloat16),
        (((1,), (1,)), ((), ())),
        preferred_element_type=jnp.float32,
    )
    part_s = jnp.sum(jnp.exp2(l2t), axis=0)

    @pl.when(j == 0)
    def _init_s():
        s_ref[...] = part_s

    @pl.when(j > 0)
    def _acc_s():
        s_ref[...] = s_ref[...] + part_s

    @pl.when(jnp.logical_and(i == 0, j == 0))
    def _init():
        acc_ref[0, 0] = jnp.float32(0.0)

    @pl.when(j == i)
    def _pos():
        acc_ref[0, 0] -= jnp.sum(u_ref[...] * pblk_ref[...])

    @pl.when(j == _B // _BLK - 1)
    def _fin():
        acc_ref[0, 0] += jnp.sum(jnp.log(s_ref[...]))


def _tc_loss(u_emb, p_emb):
    nb = _B // _BLK
    out = pl.pallas_call(
        _loss_body,
        grid=(nb, nb),
        in_specs=[
            pl.BlockSpec((_BLK, _D), lambda i, j: (i, 0)),
            pl.BlockSpec((_BLK, _D), lambda i, j: (j, 0)),
        ],
        out_specs=pl.BlockSpec(memory_space=pltpu.SMEM),
        out_shape=jax.ShapeDtypeStruct((1, 1), jnp.float32),
        scratch_shapes=[pltpu.VMEM((_BLK,), jnp.float32)],
    )(u_emb, p_emb)
    return out[0, 0]


def kernel(user_id, product_id, user_table, product_table):
    u_emb, p_emb = _sc_gather(user_id, product_id, user_table, product_table)
    return _tc_loss(u_emb, p_emb)


# speedup vs baseline: 1.0294x; 1.0294x over previous
import functools

import jax
import jax.numpy as jnp
from jax import lax
from jax.experimental import pallas as pl
from jax.experimental.pallas import tpu as pltpu
from jax.experimental.pallas import tpu_sc as plsc

_B = 4096
_D = 128
_BLK = 2048
_LOG2E = 1.4426950408889634


def _sc_gather(user_id, product_id, user_table, product_table):
    info = plsc.get_sparse_core_info()
    nw = info.num_cores * info.num_subcores
    bpw = _B // nw
    mesh = plsc.VectorSubcoreMesh(core_axis_name="c", subcore_axis_name="s")
    half = bpw // 2

    @functools.partial(
        pl.kernel,
        out_type=(
            jax.ShapeDtypeStruct((_B, _D), jnp.float32),
            jax.ShapeDtypeStruct((_B, _D), jnp.float32),
        ),
        mesh=mesh,
        scratch_types=(
            pltpu.VMEM((bpw,), jnp.int32),
            pltpu.VMEM((bpw, _D), jnp.float32),
            pltpu.VMEM((bpw,), jnp.int32),
            pltpu.VMEM((bpw, _D), jnp.float32),
            pltpu.SemaphoreType.DMA,
            pltpu.SemaphoreType.DMA,
            pltpu.SemaphoreType.DMA,
            pltpu.SemaphoreType.DMA,
        ),
    )
    def gather(uid_hbm, pid_hbm, utab_hbm, ptab_hbm, uout_hbm, pout_hbm,
               uidx, urows, pidx, prows, usem, psem, s1, s2):
        wid = lax.axis_index("s") * info.num_cores + lax.axis_index("c")
        base = wid * bpw
        ci = pltpu.async_copy(pid_hbm.at[pl.ds(base, bpw)], pidx, s1)
        cj = pltpu.async_copy(uid_hbm.at[pl.ds(base, bpw)], uidx, s2)
        ci.wait()
        cp0 = pltpu.async_copy(ptab_hbm.at[pidx.at[pl.ds(0, half)]],
                               prows.at[pl.ds(0, half)], psem)
        cp1 = pltpu.async_copy(ptab_hbm.at[pidx.at[pl.ds(half, half)]],
                               prows.at[pl.ds(half, half)], psem)
        cj.wait()
        cu0 = pltpu.async_copy(utab_hbm.at[uidx.at[pl.ds(0, half)]],
                               urows.at[pl.ds(0, half)], usem)
        cu1 = pltpu.async_copy(utab_hbm.at[uidx.at[pl.ds(half, half)]],
                               urows.at[pl.ds(half, half)], usem)
        cp0.wait()
        sp0 = pltpu.async_copy(prows.at[pl.ds(0, half)],
                               pout_hbm.at[pl.ds(base, half)], s1)
        cp1.wait()
        sp1 = pltpu.async_copy(prows.at[pl.ds(half, half)],
                               pout_hbm.at[pl.ds(base + half, half)], s2)
        cu0.wait()
        su0 = pltpu.async_copy(urows.at[pl.ds(0, half)],
                               uout_hbm.at[pl.ds(base, half)], s1)
        cu1.wait()
        su1 = pltpu.async_copy(urows.at[pl.ds(half, half)],
                               uout_hbm.at[pl.ds(base + half, half)], s2)
        sp0.wait()
        sp1.wait()
        su0.wait()
        su1.wait()

    return gather(user_id, product_id, user_table, product_table)


def _loss_body(u_ref, pall_ref, acc_ref):
    i = pl.program_id(0)
    l2t = lax.dot_general(
        pall_ref[...].astype(jnp.bfloat16),
        (u_ref[...] * _LOG2E).astype(jnp.bfloat16),
        (((1,), (1,)), ((), ())),
        preferred_element_type=jnp.float32,
    )
    s = jnp.sum(jnp.exp2(l2t), axis=0)
    pdiag = pall_ref[pl.ds(i * _BLK, _BLK), :]
    part = jnp.sum(jnp.log(s)) - jnp.sum(u_ref[...] * pdiag)

    @pl.when(i == 0)
    def _init():
        acc_ref[0, 0] = jnp.float32(0.0)

    acc_ref[0, 0] += part


def _tc_loss(u_emb, p_emb):
    out = pl.pallas_call(
        _loss_body,
        grid=(_B // _BLK,),
        in_specs=[
            pl.BlockSpec((_BLK, _D), lambda i: (i, 0)),
            pl.BlockSpec((_B, _D), lambda i: (0, 0)),
        ],
        out_specs=pl.BlockSpec(memory_space=pltpu.SMEM),
        out_shape=jax.ShapeDtypeStruct((1, 1), jnp.float32),
    )(u_emb, p_emb)
    return out[0, 0]


def kernel(user_id, product_id, user_table, product_table):
    u_emb, p_emb = _sc_gather(user_id, product_id, user_table, product_table)
    return _tc_loss(u_emb, p_emb)
